# baseline (speedup 1.0000x reference)
import functools

import jax
import jax.numpy as jnp
from jax.experimental import pallas as pl

F32 = jnp.float32
BF16 = jnp.bfloat16


def _row_expand_mat(rows, cols, dtype):
    ri = jax.lax.broadcasted_iota(jnp.int32, (rows, cols), 0)
    ci = jax.lax.broadcasted_iota(jnp.int32, (rows, cols), 1)
    return ((ri // 3) == ci).astype(dtype)


def _prep_kernel(n1, T, L, n_rel, E,
                 r_ref, start_ref, end_ref, w3_ref, wt_ref, x_ref, t2rT_ref,
                 srel0_ref, srel1_ref, srel2_ref, x48_ref, wsel_ref):
    B = x_ref.shape[0]
    BL = B * L
    N = t2rT_ref.shape[1]
    rc = jax.lax.broadcasted_iota(jnp.int32, (B, n1), 1)
    oh = (r_ref[...] == rc).astype(F32)
    col = jax.lax.broadcasted_iota(jnp.int32, (B, N), 1)
    span = (((start_ref[...] <= col) & (col < end_ref[...])) | (col < E)
            ).astype(F32)
    ebc = _row_expand_mat(BL, B, F32)
    span48 = jnp.dot(ebc, span, preferred_element_type=F32)
    x48 = jnp.dot(ebc, x_ref[...], preferred_element_type=F32)
    x48_ref[...] = x48.astype(BF16)
    ri = jax.lax.broadcasted_iota(jnp.int32, (BL, B), 0)
    ci = jax.lax.broadcasted_iota(jnp.int32, (BL, B), 1)
    srefs = (srel0_ref, srel1_ref, srel2_ref)
    for t in range(T):
        pstack = jnp.zeros((BL, n_rel), dtype=F32)
        for l in range(L):
            g = jnp.dot(oh, w3_ref[t * L + l], preferred_element_type=F32)
            g = g - jnp.max(g, axis=1, keepdims=True)
            p = jnp.exp(g)
            p = p / jnp.sum(p, axis=1, keepdims=True)
            el = ((ri - 3 * (ri // 3)) == l) & ((ri // 3) == ci)
            pstack = pstack + jnp.dot(el.astype(F32), p,
                                      preferred_element_type=F32)
        srel = jnp.dot(pstack, t2rT_ref[...], preferred_element_type=F32)
        srefs[t][...] = (srel * span48).astype(BF16)
    wts = jax.nn.sigmoid(jnp.dot(oh, wt_ref[...], preferred_element_type=F32))
    ri2 = jax.lax.broadcasted_iota(jnp.int32, (B, BL), 0)
    ci2 = jax.lax.broadcasted_iota(jnp.int32, (B, BL), 1)
    tile = ((ci2 - 3 * (ci2 // 3))[:L, :] ==
            jax.lax.broadcasted_iota(jnp.int32, (L, BL), 0)).astype(F32)
    wtile = jnp.dot(wts, tile, preferred_element_type=F32)
    wsel_ref[...] = wtile * ((ci2 // 3) == ri2).astype(F32)


def _pass0_kernel(nsteps, x48_ref, e2_ref, t2e_ref, srel_ref, s_ref):
    k = pl.program_id(0)
    xo = jnp.dot(x48_ref[...], e2_ref[...], preferred_element_type=F32)
    masked = (xo * srel_ref[...].astype(F32)).astype(BF16)
    contrib = jnp.dot(masked, t2e_ref[...], preferred_element_type=F32)

    @pl.when(k == 0)
    def _():
        s_ref[...] = jnp.zeros_like(s_ref)

    s_ref[...] += contrib

    @pl.when(k == nsteps - 1)
    def _():
        s_ref[...] = jnp.clip(s_ref[...], 0.0, 1.0)


def _passA_kernel(nsteps, sp_ref, t2e_ref, t2t_ref, c_ref):
    k = pl.program_id(0)
    sp = sp_ref[...].astype(BF16)
    xp = jax.lax.dot_general(sp, t2e_ref[...],
                             (((1,), (1,)), ((), ())),
                             preferred_element_type=F32)
    sgn = ((xp > 0.0).astype(F32) - (xp < 0.0).astype(F32)).astype(BF16)
    contrib = jnp.dot(sgn, t2t_ref[...], preferred_element_type=F32)

    @pl.when(k == 0)
    def _():
        c_ref[...] = jnp.zeros_like(c_ref)

    c_ref[...] += contrib


def _passB_kernel(nsteps, bN, E, final,
                  sp_ref, c_ref, e2_ref, t2e_ref, t2tT_ref, srel_ref,
                  *rest):
    if final:
        wsel_ref, s_ref, out_ref = rest
    else:
        (s_ref,) = rest
    k = pl.program_id(0)
    m = c_ref.shape[1]
    sp = sp_ref[...].astype(BF16)
    x = jnp.dot(sp, e2_ref[...], preferred_element_type=F32)
    ti = jax.lax.broadcasted_iota(jnp.int32, (m, m), 0)
    tj = jax.lax.broadcasted_iota(jnp.int32, (m, m), 1)
    triu = (ti <= tj).astype(F32)
    c2 = jnp.dot(c_ref[...], triu, preferred_element_type=F32).astype(BF16)
    mv = jnp.dot(c2, t2tT_ref[...], preferred_element_type=F32)
    mv = jnp.clip(mv, 0.0, 1.0)
    col = k * bN + jax.lax.broadcasted_iota(jnp.int32, x.shape, 1)
    mv = jnp.where(col < E, 1.0, mv)
    masked = (x * mv * srel_ref[...].astype(F32)).astype(BF16)
    contrib = jnp.dot(masked, t2e_ref[...], preferred_element_type=F32)

    @pl.when(k == 0)
    def _():
        s_ref[...] = jnp.zeros_like(s_ref)

    s_ref[...] += contrib

    @pl.when(k == nsteps - 1)
    def _():
        sc = jnp.clip(s_ref[...], 0.0, 1.0)
        s_ref[...] = sc
        if final:
            out_ref[...] = jnp.dot(wsel_ref[...], sc,
                                   preferred_element_type=F32)


def kernel(input_x, input_r, input_triple2id, e2triple, triple2e, triple2r,
           triple2time, w_params, weight_params):
    B, E = input_x.shape
    N = triple2e.shape[0]
    m = triple2time.shape[1]
    n1, T, L, n_rel = w_params.shape
    BL = B * L

    bN = 2048
    K = -(-N // bN)
    Np = K * bN
    padN = Np - N

    e2_bf = jnp.pad(e2triple, ((0, 0), (0, padN))).astype(BF16)
    t2e_bf = jnp.pad(triple2e, ((0, padN), (0, 0))).astype(BF16)
    t2t_bf = jnp.pad(triple2time, ((0, padN), (0, 0))).astype(BF16)
    t2tT_bf = jnp.pad(triple2time.T, ((0, 0), (0, padN))).astype(BF16)
    t2rT = jnp.pad(triple2r.T, ((0, 0), (0, padN)))
    w3 = jnp.transpose(w_params.reshape(n1, T * L, n_rel), (1, 0, 2))
    wt2 = weight_params[..., 0]
    r2 = input_r.astype(jnp.int32).reshape(B, 1)
    t2id = input_triple2id.astype(jnp.int32)
    start = t2id[:, 0:1]
    end = t2id[:, 1:2]

    srel0, srel1, srel2, x48, wsel = pl.pallas_call(
        functools.partial(_prep_kernel, n1, T, L, n_rel, E),
        out_shape=[
            jax.ShapeDtypeStruct((BL, Np), BF16),
            jax.ShapeDtypeStruct((BL, Np), BF16),
            jax.ShapeDtypeStruct((BL, Np), BF16),
            jax.ShapeDtypeStruct((BL, E), BF16),
            jax.ShapeDtypeStruct((B, BL), F32),
        ],
    )(r2, start, end, w3, wt2, input_x, t2rT)

    s0 = pl.pallas_call(
        functools.partial(_pass0_kernel, K),
        grid=(K,),
        in_specs=[
            pl.BlockSpec((BL, E), lambda k: (0, 0)),
            pl.BlockSpec((E, bN), lambda k: (0, k)),
            pl.BlockSpec((bN, E), lambda k: (k, 0)),
            pl.BlockSpec((BL, bN), lambda k: (0, k)),
        ],
        out_specs=pl.BlockSpec((BL, E), lambda k: (0, 0)),
        out_shape=jax.ShapeDtypeStruct((BL, E), F32),
    )(x48, e2_bf, t2e_bf, srel0)

    def run_passA(sp):
        return pl.pallas_call(
            functools.partial(_passA_kernel, K),
            grid=(K,),
            in_specs=[
                pl.BlockSpec((BL, E), lambda k: (0, 0)),
                pl.BlockSpec((bN, E), lambda k: (k, 0)),
                pl.BlockSpec((bN, m), lambda k: (k, 0)),
            ],
            out_specs=pl.BlockSpec((BL, m), lambda k: (0, 0)),
            out_shape=jax.ShapeDtypeStruct((BL, m), F32),
        )(sp, t2e_bf, t2t_bf)

    def run_passB(sp, c, srel, final):
        in_specs = [
            pl.BlockSpec((BL, E), lambda k: (0, 0)),
            pl.BlockSpec((BL, m), lambda k: (0, 0)),
            pl.BlockSpec((E, bN), lambda k: (0, k)),
            pl.BlockSpec((bN, E), lambda k: (k, 0)),
            pl.BlockSpec((m, bN), lambda k: (0, k)),
            pl.BlockSpec((BL, bN), lambda k: (0, k)),
        ]
        args = [sp, c, e2_bf, t2e_bf, t2tT_bf, srel]
        out_specs = [pl.BlockSpec((BL, E), lambda k: (0, 0))]
        out_shape = [jax.ShapeDtypeStruct((BL, E), F32)]
        if final:
            in_specs.append(pl.BlockSpec((B, BL), lambda k: (0, 0)))
            args.append(wsel)
            out_specs.append(pl.BlockSpec((B, E), lambda k: (0, 0)))
            out_shape.append(jax.ShapeDtypeStruct((B, E), F32))
        res = pl.pallas_call(
            functools.partial(_passB_kernel, K, bN, E, final),
            grid=(K,),
            in_specs=in_specs,
            out_specs=out_specs,
            out_shape=out_shape,
        )(*args)
        return res

    c1 = run_passA(s0)
    (s1,) = run_passB(s0, c1, srel1, final=False)
    c2 = run_passA(s1)
    _, out = run_passB(s1, c2, srel2, final=True)
    return out


# speedup vs baseline: 1.5879x; 1.5879x over previous
import functools

import jax
import jax.numpy as jnp
from jax.experimental import pallas as pl

F32 = jnp.float32
BF16 = jnp.bfloat16
F8 = jnp.float8_e4m3fn
SE = 1024.0
ST = 4096.0
SP = 256.0
SM = 128.0


def _row_expand_mat(rows, cols, dtype):
    ri = jax.lax.broadcasted_iota(jnp.int32, (rows, cols), 0)
    ci = jax.lax.broadcasted_iota(jnp.int32, (rows, cols), 1)
    return ((ri // 3) == ci).astype(dtype)


def _prep_kernel(n1, T, L, n_rel, E,
                 r_ref, start_ref, end_ref, w3_ref, wt_ref, x_ref, t2rT_ref,
                 srel0_ref, srel1_ref, srel2_ref, x48_ref, wsel_ref):
    B = x_ref.shape[0]
    BL = B * L
    N = t2rT_ref.shape[1]
    rc = jax.lax.broadcasted_iota(jnp.int32, (B, n1), 1)
    oh = (r_ref[...] == rc).astype(F32)
    col = jax.lax.broadcasted_iota(jnp.int32, (B, N), 1)
    span = (((start_ref[...] <= col) & (col < end_ref[...])) | (col < E)
            ).astype(F32)
    ebc = _row_expand_mat(BL, B, F32)
    span48 = jnp.dot(ebc, span, preferred_element_type=F32)
    x48 = jnp.dot(ebc, x_ref[...], preferred_element_type=F32)
    x48_ref[...] = x48.astype(BF16)
    ri = jax.lax.broadcasted_iota(jnp.int32, (BL, B), 0)
    ci = jax.lax.broadcasted_iota(jnp.int32, (BL, B), 1)
    srefs = (srel0_ref, srel1_ref, srel2_ref)
    for t in range(T):
        pstack = jnp.zeros((BL, n_rel), dtype=F32)
        for l in range(L):
            g = jnp.dot(oh, w3_ref[t * L + l], preferred_element_type=F32)
            g = g - jnp.max(g, axis=1, keepdims=True)
            p = jnp.exp(g)
            p = p / jnp.sum(p, axis=1, keepdims=True)
            el = ((ri - 3 * (ri // 3)) == l) & ((ri // 3) == ci)
            pstack = pstack + jnp.dot(el.astype(F32), p,
                                      preferred_element_type=F32)
        srel = jnp.dot(pstack, t2rT_ref[...], preferred_element_type=F32)
        srefs[t][...] = (srel * span48).astype(BF16)
    wts = jax.nn.sigmoid(jnp.dot(oh, wt_ref[...], preferred_element_type=F32))
    ri2 = jax.lax.broadcasted_iota(jnp.int32, (B, BL), 0)
    ci2 = jax.lax.broadcasted_iota(jnp.int32, (B, BL), 1)
    tile = ((ci2 - 3 * (ci2 // 3))[:L, :] ==
            jax.lax.broadcasted_iota(jnp.int32, (L, BL), 0)).astype(F32)
    wtile = jnp.dot(wts, tile, preferred_element_type=F32)
    wsel_ref[...] = wtile * ((ci2 // 3) == ri2).astype(F32)


def _pass0_kernel(nsteps, bN, N,
                  x48_ref, e2f_ref, t2ef_ref, srel_ref,
                  s_ref, e2q_ref, t2eq_ref):
    k = pl.program_id(0)
    colv = k * bN + jax.lax.broadcasted_iota(jnp.int32, e2f_ref.shape, 1)
    e2q = jnp.where(colv < N, e2f_ref[...] * SE, 0.0).astype(F8)
    e2q_ref[...] = e2q
    rowv = k * bN + jax.lax.broadcasted_iota(jnp.int32, t2ef_ref.shape, 0)
    t2eq = jnp.where(rowv < N, t2ef_ref[...] * ST, 0.0).astype(F8)
    t2eq_ref[...] = t2eq

    xo = jnp.dot(x48_ref[...], e2q, preferred_element_type=F32) * (1.0 / SE)
    masked = (xo * srel_ref[...].astype(F32)).astype(BF16)
    contrib = jnp.dot(masked, t2eq, preferred_element_type=F32)

    @pl.when(k == 0)
    def _():
        s_ref[...] = jnp.zeros_like(s_ref)

    s_ref[...] += contrib

    @pl.when(k == nsteps - 1)
    def _():
        s_ref[...] = jnp.clip(s_ref[...] * (1.0 / ST), 0.0, 1.0)


def _passA_kernel(nsteps, sp_ref, t2eq_ref, t2t_ref, c_ref):
    k = pl.program_id(0)
    sp = sp_ref[...].astype(BF16)
    xp = jax.lax.dot_general(sp, t2eq_ref[...],
                             (((1,), (1,)), ((), ())),
                             preferred_element_type=F32)
    sgn = ((xp > 0.0).astype(F32) - (xp < 0.0).astype(F32)).astype(BF16)
    contrib = jnp.dot(sgn, t2t_ref[...], preferred_element_type=F32)

    @pl.when(k == 0)
    def _():
        c_ref[...] = jnp.zeros_like(c_ref)

    c_ref[...] += contrib


def _passB_kernel(nsteps, bN, E, final,
                  sp_ref, c_ref, e2q_ref, t2eq_ref, t2tT_ref, srel_ref,
                  *rest):
    if final:
        wsel_ref, s_ref, out_ref = rest
    else:
        (s_ref,) = rest
    k = pl.program_id(0)
    m = c_ref.shape[1]
    sp = sp_ref[...].astype(BF16)
    x = jnp.dot(sp, e2q_ref[...], preferred_element_type=F32) * (1.0 / SE)
    ti = jax.lax.broadcasted_iota(jnp.int32, (m, m), 0)
    tj = jax.lax.broadcasted_iota(jnp.int32, (m, m), 1)
    triu = (ti <= tj).astype(F32)
    c2 = jnp.dot(c_ref[...], triu, preferred_element_type=F32).astype(BF16)
    mv = jnp.dot(c2, t2tT_ref[...], preferred_element_type=F32)
    mv = jnp.clip(mv, 0.0, 1.0)
    col = k * bN + jax.lax.broadcasted_iota(jnp.int32, x.shape, 1)
    mv = jnp.where(col < E, 1.0, mv)
    masked = (x * mv * srel_ref[...].astype(F32)).astype(BF16)
    contrib = jnp.dot(masked, t2eq_ref[...], preferred_element_type=F32)

    @pl.when(k == 0)
    def _():
        s_ref[...] = jnp.zeros_like(s_ref)

    s_ref[...] += contrib

    @pl.when(k == nsteps - 1)
    def _():
        sc = jnp.clip(s_ref[...] * (1.0 / ST), 0.0, 1.0)
        s_ref[...] = sc
        if final:
            out_ref[...] = jnp.dot(wsel_ref[...], sc,
                                   preferred_element_type=F32)


def kernel(input_x, input_r, input_triple2id, e2triple, triple2e, triple2r,
           triple2time, w_params, weight_params):
    B, E = input_x.shape
    N = triple2e.shape[0]
    m = triple2time.shape[1]
    n1, T, L, n_rel = w_params.shape
    BL = B * L

    bN0 = 1024
    K0 = -(-N // bN0)
    Np = K0 * bN0
    bN = 2048
    K = Np // bN

    t2t_bf = jnp.pad(triple2time, ((0, Np - N), (0, 0))).astype(BF16)
    t2tT_bf = jnp.pad(triple2time.T, ((0, 0), (0, Np - N))).astype(BF16)
    t2rT = jnp.pad(triple2r.T, ((0, 0), (0, Np - N)))
    w3 = jnp.transpose(w_params.reshape(n1, T * L, n_rel), (1, 0, 2))
    wt2 = weight_params[..., 0]
    r2 = input_r.astype(jnp.int32).reshape(B, 1)
    t2id = input_triple2id.astype(jnp.int32)
    start = t2id[:, 0:1]
    end = t2id[:, 1:2]

    srel0, srel1, srel2, x48q, wsel = pl.pallas_call(
        functools.partial(_prep_kernel, n1, T, L, n_rel, E),
        out_shape=[
            jax.ShapeDtypeStruct((BL, Np), BF16),
            jax.ShapeDtypeStruct((BL, Np), BF16),
            jax.ShapeDtypeStruct((BL, Np), BF16),
            jax.ShapeDtypeStruct((BL, E), BF16),
            jax.ShapeDtypeStruct((B, BL), F32),
        ],
    )(r2, start, end, w3, wt2, input_x, t2rT)

    s0, e2q, t2eq = pl.pallas_call(
        functools.partial(_pass0_kernel, K0, bN0, N),
        grid=(K0,),
        in_specs=[
            pl.BlockSpec((BL, E), lambda k: (0, 0)),
            pl.BlockSpec((E, bN0), lambda k: (0, k)),
            pl.BlockSpec((bN0, E), lambda k: (k, 0)),
            pl.BlockSpec((BL, bN0), lambda k: (0, k)),
        ],
        out_specs=[
            pl.BlockSpec((BL, E), lambda k: (0, 0)),
            pl.BlockSpec((E, bN0), lambda k: (0, k)),
            pl.BlockSpec((bN0, E), lambda k: (k, 0)),
        ],
        out_shape=[
            jax.ShapeDtypeStruct((BL, E), F32),
            jax.ShapeDtypeStruct((E, Np), F8),
            jax.ShapeDtypeStruct((Np, E), F8),
        ],
    )(x48q, e2triple, triple2e, srel0)

    def run_passA(sp):
        return pl.pallas_call(
            functools.partial(_passA_kernel, K),
            grid=(K,),
            in_specs=[
                pl.BlockSpec((BL, E), lambda k: (0, 0)),
                pl.BlockSpec((bN, E), lambda k: (k, 0)),
                pl.BlockSpec((bN, m), lambda k: (k, 0)),
            ],
            out_specs=pl.BlockSpec((BL, m), lambda k: (0, 0)),
            out_shape=jax.ShapeDtypeStruct((BL, m), F32),
        )(sp, t2eq, t2t_bf)

    def run_passB(sp, c, srel, final):
        in_specs = [
            pl.BlockSpec((BL, E), lambda k: (0, 0)),
            pl.BlockSpec((BL, m), lambda k: (0, 0)),
            pl.BlockSpec((E, bN), lambda k: (0, k)),
            pl.BlockSpec((bN, E), lambda k: (k, 0)),
            pl.BlockSpec((m, bN), lambda k: (0, k)),
            pl.BlockSpec((BL, bN), lambda k: (0, k)),
        ]
        args = [sp, c, e2q, t2eq, t2tT_bf, srel]
        out_specs = [pl.BlockSpec((BL, E), lambda k: (0, 0))]
        out_shape = [jax.ShapeDtypeStruct((BL, E), F32)]
        if final:
            in_specs.append(pl.BlockSpec((B, BL), lambda k: (0, 0)))
            args.append(wsel)
            out_specs.append(pl.BlockSpec((B, E), lambda k: (0, 0)))
            out_shape.append(jax.ShapeDtypeStruct((B, E), F32))
        return pl.pallas_call(
            functools.partial(_passB_kernel, K, bN, E, final),
            grid=(K,),
            in_specs=in_specs,
            out_specs=out_specs,
            out_shape=out_shape,
        )(*args)

    c1 = run_passA(s0)
    (s1,) = run_passB(s0, c1, srel1, final=False)
    c2 = run_passA(s1)
    _, out = run_passB(s1, c2, srel2, final=True)
    return out


# speedup vs baseline: 1.7603x; 1.1086x over previous
import functools

import jax
import jax.numpy as jnp
from jax.experimental import pallas as pl
from jax.experimental.pallas import tpu as pltpu

F32 = jnp.float32
BF16 = jnp.bfloat16
F8 = jnp.float8_e4m3fn
SE = 1024.0
ST = 4096.0


def _mega_kernel(n1, T, L, n_rel, E, N, Np, bN0, P0, bN, CH,
                 r_ref, start_ref, end_ref, w3_ref, wt_ref, x_ref, t2rT_ref,
                 e2f_ref, t2ef_ref, t2tq_ref, t2tTq_ref,
                 out_ref,
                 e2q_ref, t2eq_ref, srel_ref, x48_ref, wsel_ref,
                 sa_ref, sb_ref, c_ref):
    i = pl.program_id(0)
    B = x_ref.shape[0]
    BL = B * L
    m = t2tq_ref.shape[1]

    @pl.when(i == 0)
    def _prep():
        rc = jax.lax.broadcasted_iota(jnp.int32, (B, n1), 1)
        oh = (r_ref[...] == rc).astype(F32)
        col = jax.lax.broadcasted_iota(jnp.int32, (B, Np), 1)
        span = (((start_ref[...] <= col) & (col < end_ref[...])) | (col < E)
                ).astype(F32)
        ri = jax.lax.broadcasted_iota(jnp.int32, (BL, B), 0)
        ci = jax.lax.broadcasted_iota(jnp.int32, (BL, B), 1)
        ebc = ((ri // 3) == ci).astype(F32)
        span48 = jnp.dot(ebc, span, preferred_element_type=F32)
        x48_ref[...] = jnp.dot(ebc, x_ref[...],
                               preferred_element_type=F32).astype(BF16)
        for t in range(T):
            pstack = jnp.zeros((BL, n_rel), dtype=F32)
            for l in range(L):
                g = jnp.dot(oh, w3_ref[t * L + l], preferred_element_type=F32)
                g = g - jnp.max(g, axis=1, keepdims=True)
                p = jnp.exp(g)
                p = p / jnp.sum(p, axis=1, keepdims=True)
                el = ((ri - 3 * (ri // 3)) == l) & ((ri // 3) == ci)
                pstack = pstack + jnp.dot(el.astype(F32), p,
                                          preferred_element_type=F32)
            srel = jnp.dot(pstack, t2rT_ref[...], preferred_element_type=F32)
            srel_ref[t * BL:(t + 1) * BL, :] = (srel * span48).astype(BF16)
        wts = jax.nn.sigmoid(jnp.dot(oh, wt_ref[...],
                                     preferred_element_type=F32))
        ri2 = jax.lax.broadcasted_iota(jnp.int32, (B, BL), 0)
        ci2 = jax.lax.broadcasted_iota(jnp.int32, (B, BL), 1)
        tile = ((ci2 - 3 * (ci2 // 3))[:L, :] ==
                jax.lax.broadcasted_iota(jnp.int32, (L, BL), 0)).astype(F32)
        wtile = jnp.dot(wts, tile, preferred_element_type=F32)
        wsel_ref[...] = wtile * ((ci2 // 3) == ri2).astype(F32)
        sa_ref[...] = jnp.zeros_like(sa_ref)

    @pl.when(i < P0)
    def _load_quant_hop0():
        sl = pl.ds(i * bN0, bN0)
        colv = i * bN0 + jax.lax.broadcasted_iota(jnp.int32, e2f_ref.shape, 1)
        e2q = jnp.where(colv < N, e2f_ref[...] * SE, 0.0).astype(F8)
        e2q_ref[:, sl] = e2q
        rowv = i * bN0 + jax.lax.broadcasted_iota(jnp.int32, t2ef_ref.shape, 0)
        t2eq = jnp.where(rowv < N, t2ef_ref[...] * ST, 0.0).astype(F8)
        t2eq_ref[sl, :] = t2eq
        xo = jnp.dot(x48_ref[...], e2q, preferred_element_type=F32) * (1.0 / SE)
        masked = (xo * srel_ref[0:BL, sl].astype(F32)).astype(BF16)
        sa_ref[...] += jnp.dot(masked, t2eq, preferred_element_type=F32)

    @pl.when(i == P0 - 1)
    def _finish_hop0():
        sa_ref[...] = jnp.clip(sa_ref[...] * (1.0 / ST), 0.0, 1.0)

    def phase_a(j, sp_ref):
        sl = pl.ds(j * bN, bN)
        sp = sp_ref[...].astype(BF16)
        xp = jax.lax.dot_general(sp, t2eq_ref[sl, :],
                                 (((1,), (1,)), ((), ())),
                                 preferred_element_type=F32)
        sgn = ((xp > 0.0).astype(F32) - (xp < 0.0).astype(F32)).astype(BF16)
        contrib = jnp.dot(sgn, t2tq_ref[sl, :], preferred_element_type=F32)

        @pl.when(j == 0)
        def _():
            c_ref[...] = jnp.zeros_like(c_ref)

        c_ref[...] += contrib

    def phase_b(j, sp_ref, acc_ref, srow, final):
        sl = pl.ds(j * bN, bN)
        sp = sp_ref[...].astype(BF16)
        x = jnp.dot(sp, e2q_ref[:, sl],
                    preferred_element_type=F32) * (1.0 / SE)
        ti = jax.lax.broadcasted_iota(jnp.int32, (m, m), 0)
        tj = jax.lax.broadcasted_iota(jnp.int32, (m, m), 1)
        triu = (ti <= tj).astype(F32)
        c2 = jnp.dot(c_ref[...], triu, preferred_element_type=F32).astype(BF16)
        mv = jnp.dot(c2, t2tTq_ref[:, sl], preferred_element_type=F32)
        mv = jnp.clip(mv, 0.0, 1.0)
        col = j * bN + jax.lax.broadcasted_iota(jnp.int32, x.shape, 1)
        mv = jnp.where(col < E, 1.0, mv)
        masked = (x * mv * srel_ref[srow:srow + BL, sl].astype(F32)
                  ).astype(BF16)
        contrib = jnp.dot(masked, t2eq_ref[sl, :], preferred_element_type=F32)

        @pl.when(j == 0)
        def _():
            acc_ref[...] = jnp.zeros_like(acc_ref)

        acc_ref[...] += contrib

        @pl.when(j == CH - 1)
        def _():
            sc = jnp.clip(acc_ref[...] * (1.0 / ST), 0.0, 1.0)
            acc_ref[...] = sc
            if final:
                out_ref[...] = jnp.dot(wsel_ref[...], sc,
                                       preferred_element_type=F32)

    @pl.when((i >= P0) & (i < P0 + CH))
    def _a1():
        phase_a(i - P0, sa_ref)

    @pl.when((i >= P0 + CH) & (i < P0 + 2 * CH))
    def _b1():
        phase_b(i - (P0 + CH), sa_ref, sb_ref, BL, False)

    @pl.when((i >= P0 + 2 * CH) & (i < P0 + 3 * CH))
    def _a2():
        phase_a(i - (P0 + 2 * CH), sb_ref)

    @pl.when((i >= P0 + 3 * CH) & (i < P0 + 4 * CH))
    def _b2():
        phase_b(i - (P0 + 3 * CH), sb_ref, sa_ref, 2 * BL, True)


def kernel(input_x, input_r, input_triple2id, e2triple, triple2e, triple2r,
           triple2time, w_params, weight_params):
    B, E = input_x.shape
    N = triple2e.shape[0]
    m = triple2time.shape[1]
    n1, T, L, n_rel = w_params.shape
    BL = B * L

    bN0 = 256
    P0 = -(-N // bN0)
    Np = P0 * bN0
    bN = 2048
    CH = Np // bN
    nsteps = P0 + 4 * CH

    t2tq = jnp.pad(triple2time, ((0, Np - N), (0, 0))).astype(F8)
    t2tTq = jnp.pad(triple2time.T, ((0, 0), (0, Np - N))).astype(F8)
    t2rT = jnp.pad(triple2r.T, ((0, 0), (0, Np - N)))
    w3 = jnp.transpose(w_params.reshape(n1, T * L, n_rel), (1, 0, 2))
    wt2 = weight_params[..., 0]
    r2 = input_r.astype(jnp.int32).reshape(B, 1)
    t2id = input_triple2id.astype(jnp.int32)
    start = t2id[:, 0:1]
    end = t2id[:, 1:2]

    c0 = lambda i: (0, 0)
    out = pl.pallas_call(
        functools.partial(_mega_kernel, n1, T, L, n_rel, E, N, Np,
                          bN0, P0, bN, CH),
        grid=(nsteps,),
        in_specs=[
            pl.BlockSpec((B, 1), c0),
            pl.BlockSpec((B, 1), c0),
            pl.BlockSpec((B, 1), c0),
            pl.BlockSpec((T * L, n1, n_rel), lambda i: (0, 0, 0)),
            pl.BlockSpec((n1, L), c0),
            pl.BlockSpec((B, E), c0),
            pl.BlockSpec((n_rel, Np), c0),
            pl.BlockSpec((E, bN0), lambda i: (0, jnp.minimum(i, P0 - 1))),
            pl.BlockSpec((bN0, E), lambda i: (jnp.minimum(i, P0 - 1), 0)),
            pl.BlockSpec((Np, m), c0),
            pl.BlockSpec((m, Np), c0),
        ],
        out_specs=pl.BlockSpec((B, E), c0),
        out_shape=jax.ShapeDtypeStruct((B, E), F32),
        scratch_shapes=[
            pltpu.VMEM((E, Np), F8),
            pltpu.VMEM((Np, E), F8),
            pltpu.VMEM((T * BL, Np), BF16),
            pltpu.VMEM((BL, E), BF16),
            pltpu.VMEM((B, BL), F32),
            pltpu.VMEM((BL, E), F32),
            pltpu.VMEM((BL, E), F32),
            pltpu.VMEM((BL, m), F32),
        ],
    )(r2, start, end, w3, wt2, input_x, t2rT,
      e2triple, triple2e, t2tq, t2tTq)
    return out


# speedup vs baseline: 2.0405x; 1.1591x over previous
import functools

import jax
import jax.numpy as jnp
from jax.experimental import pallas as pl
from jax.experimental.pallas import tpu as pltpu

F32 = jnp.float32
BF16 = jnp.bfloat16
F8 = jnp.float8_e4m3fn
SE = 1024.0
ST = 4096.0


def _mega_kernel(n1, T, L, n_rel, E, N, Np, bN0, P0, bN, CH,
                 ids_ref, w3_ref, wt_ref, x_ref, t2rT_ref,
                 e2f_ref, t2ef_ref, t2tq_ref,
                 out_ref,
                 e2q_ref, t2eq_ref, pstack_ref, se48_ref,
                 x48_ref, wsel_ref, sacc_ref, scur_ref, c_ref):
    i = pl.program_id(0)
    B = x_ref.shape[0]
    BL = B * L
    m = t2tq_ref.shape[1]

    @pl.when(i == 0)
    def _prep():
        rc = jax.lax.broadcasted_iota(jnp.int32, (B, n1), 1)
        oh = (ids_ref[:, 0:1] == rc).astype(F32)
        ri = jax.lax.broadcasted_iota(jnp.int32, (BL, B), 0)
        ci = jax.lax.broadcasted_iota(jnp.int32, (BL, B), 1)
        ebc = ((ri // 3) == ci).astype(F32)
        se48_ref[...] = jnp.dot(ebc, ids_ref[:, 1:3].astype(F32),
                                preferred_element_type=F32)
        x48_ref[...] = jnp.dot(ebc.astype(BF16), x_ref[...],
                               preferred_element_type=F32).astype(F8)
        for t in range(T):
            pstack = jnp.zeros((BL, n_rel), dtype=F32)
            for l in range(L):
                g = jnp.dot(oh.astype(BF16), w3_ref[t * L + l], preferred_element_type=F32)
                g = g - jnp.max(g, axis=1, keepdims=True)
                p = jnp.exp(g)
                p = p / jnp.sum(p, axis=1, keepdims=True)
                el = ((ri - 3 * (ri // 3)) == l) & ((ri // 3) == ci)
                pstack = pstack + jnp.dot(el.astype(F32), p,
                                          preferred_element_type=F32)
            pstack_ref[t * BL:(t + 1) * BL, :] = pstack.astype(BF16)
        wts = jax.nn.sigmoid(jnp.dot(oh, wt_ref[...],
                                     preferred_element_type=F32))
        ri2 = jax.lax.broadcasted_iota(jnp.int32, (B, BL), 0)
        ci2 = jax.lax.broadcasted_iota(jnp.int32, (B, BL), 1)
        tile = ((ci2 - 3 * (ci2 // 3))[:L, :] ==
                jax.lax.broadcasted_iota(jnp.int32, (L, BL), 0)).astype(F32)
        wtile = jnp.dot(wts, tile, preferred_element_type=F32)
        wsel_ref[...] = wtile * ((ci2 // 3) == ri2).astype(F32)
        sacc_ref[...] = jnp.zeros_like(sacc_ref)

    def srel_chunk(base, width, srow):
        ps = pstack_ref[srow:srow + BL, :]
        sr = jnp.dot(ps, t2rT_ref[:, pl.ds(base, width)],
                     preferred_element_type=F32)
        col = (base +
               jax.lax.broadcasted_iota(jnp.int32, (BL, width), 1)).astype(F32)
        span = (((se48_ref[:, 0:1] <= col) & (col < se48_ref[:, 1:2])) |
                (col < float(E))).astype(F32)
        return sr * span

    @pl.when(i < P0)
    def _load_quant_hop0():
        sl = pl.ds(i * bN0, bN0)

        @pl.when(i < P0 - 1)
        def _full_slab():
            e2q_ref[:, sl] = (e2f_ref[...] * SE).astype(F8)
            t2eq_ref[sl, :] = (t2ef_ref[...] * ST).astype(F8)

        @pl.when(i == P0 - 1)
        def _partial_slab():
            colv = (i * bN0 +
                    jax.lax.broadcasted_iota(jnp.int32, e2f_ref.shape, 1))
            e2q_ref[:, sl] = jnp.where(colv < N, e2f_ref[...] * SE,
                                       0.0).astype(F8)
            rowv = (i * bN0 +
                    jax.lax.broadcasted_iota(jnp.int32, t2ef_ref.shape, 0))
            t2eq_ref[sl, :] = jnp.where(rowv < N, t2ef_ref[...] * ST,
                                        0.0).astype(F8)

        e2q = e2q_ref[:, sl]
        t2eq = t2eq_ref[sl, :]
        xo = jnp.dot(x48_ref[...], e2q, preferred_element_type=F32) * (1.0 / SE)
        masked = (xo * srel_chunk(i * bN0, bN0, 0)).astype(BF16)
        sacc_ref[...] += jnp.dot(masked, t2eq, preferred_element_type=F32)

    @pl.when(i == P0 - 1)
    def _finish_hop0():
        scur_ref[...] = jnp.clip(sacc_ref[...] * (1.0 / ST),
                                 0.0, 1.0).astype(BF16)

    def phase_a(j):
        sl = pl.ds(j * bN, bN)
        sp = scur_ref[...]
        xp = jax.lax.dot_general(sp, t2eq_ref[sl, :],
                                 (((1,), (1,)), ((), ())),
                                 preferred_element_type=F32)
        sgn = ((xp > 0.0).astype(F32) - (xp < 0.0).astype(F32)).astype(BF16)
        contrib = jnp.dot(sgn, t2tq_ref[...], preferred_element_type=F32)

        @pl.when(j == 0)
        def _():
            c_ref[...] = jnp.zeros_like(c_ref)

        c_ref[...] += contrib

    def phase_b(j, srow, final):
        sl = pl.ds(j * bN, bN)
        sp = scur_ref[...]
        x = jnp.dot(sp, e2q_ref[:, sl],
                    preferred_element_type=F32) * (1.0 / SE)
        ti = jax.lax.broadcasted_iota(jnp.int32, (m, m), 0)
        tj = jax.lax.broadcasted_iota(jnp.int32, (m, m), 1)
        triu = (ti <= tj).astype(F32)
        c2 = jnp.dot(c_ref[...], triu, preferred_element_type=F32).astype(BF16)
        mv = jax.lax.dot_general(c2, t2tq_ref[...],
                                 (((1,), (1,)), ((), ())),
                                 preferred_element_type=F32)
        mv = jnp.clip(mv, 0.0, 1.0)
        col = j * bN + jax.lax.broadcasted_iota(jnp.int32, x.shape, 1)
        mv = jnp.where(col < E, 1.0, mv)
        masked = (x * mv * srel_chunk(j * bN, bN, srow)).astype(BF16)
        contrib = jnp.dot(masked, t2eq_ref[sl, :], preferred_element_type=F32)

        @pl.when(j == 0)
        def _():
            sacc_ref[...] = jnp.zeros_like(sacc_ref)

        sacc_ref[...] += contrib

        @pl.when(j == CH - 1)
        def _():
            sc = jnp.clip(sacc_ref[...] * (1.0 / ST), 0.0, 1.0)
            scur_ref[...] = sc.astype(BF16)
            if final:
                out_ref[...] = jnp.dot(wsel_ref[...], sc,
                                       preferred_element_type=F32)

    @pl.when((i >= P0) & (i < P0 + CH))
    def _a1():
        phase_a(i - P0)

    @pl.when((i >= P0 + CH) & (i < P0 + 2 * CH))
    def _b1():
        phase_b(i - (P0 + CH), BL, False)

    @pl.when((i >= P0 + 2 * CH) & (i < P0 + 3 * CH))
    def _a2():
        phase_a(i - (P0 + 2 * CH))

    @pl.when((i >= P0 + 3 * CH) & (i < P0 + 4 * CH))
    def _b2():
        phase_b(i - (P0 + 3 * CH), 2 * BL, True)


def kernel(input_x, input_r, input_triple2id, e2triple, triple2e, triple2r,
           triple2time, w_params, weight_params):
    B, E = input_x.shape
    N = triple2e.shape[0]
    m = triple2time.shape[1]
    n1, T, L, n_rel = w_params.shape
    BL = B * L

    bN0 = 512
    P0 = -(-N // bN0)
    Np = P0 * bN0
    bN = 2048
    CH = Np // bN
    nsteps = P0 + 4 * CH

    t2tq = jnp.pad(triple2time, ((0, Np - N), (0, 0))).astype(F8)
    t2rT = jnp.pad(triple2r.T, ((0, 0), (0, Np - N))).astype(BF16)
    w3 = jnp.transpose(w_params.reshape(n1, T * L, n_rel), (1, 0, 2)).astype(BF16)
    wt2 = weight_params[..., 0]
    ids = jnp.concatenate([
        input_r.astype(jnp.int32).reshape(B, 1),
        input_triple2id.astype(jnp.int32),
        jnp.zeros((B, 1), jnp.int32)], axis=1)

    c0 = lambda i: (0, 0)
    out = pl.pallas_call(
        functools.partial(_mega_kernel, n1, T, L, n_rel, E, N, Np,
                          bN0, P0, bN, CH),
        grid=(nsteps,),
        in_specs=[
            pl.BlockSpec((B, 4), c0),
            pl.BlockSpec((T * L, n1, n_rel), lambda i: (0, 0, 0)),
            pl.BlockSpec((n1, L), c0),
            pl.BlockSpec((B, E), c0),
            pl.BlockSpec((n_rel, Np), c0),
            pl.BlockSpec((E, bN0), lambda i: (0, jnp.minimum(i, P0 - 1))),
            pl.BlockSpec((bN0, E), lambda i: (jnp.minimum(i, P0 - 1), 0)),
            pl.BlockSpec((bN, m),
                         lambda i: (jnp.where(i < P0, 0,
                                              jax.lax.rem(i - P0, CH)), 0)),
        ],
        out_specs=pl.BlockSpec((B, E), c0),
        out_shape=jax.ShapeDtypeStruct((B, E), F32),
        scratch_shapes=[
            pltpu.VMEM((E, Np), F8),
            pltpu.VMEM((Np, E), F8),
            pltpu.VMEM((T * BL, n_rel), BF16),
            pltpu.VMEM((BL, 2), F32),
            pltpu.VMEM((BL, E), F8),
            pltpu.VMEM((B, BL), F32),
            pltpu.VMEM((BL, E), F32),
            pltpu.VMEM((BL, E), BF16),
            pltpu.VMEM((BL, m), F32),
        ],
    )(ids, w3, wt2, input_x.astype(BF16), t2rT,
      e2triple, triple2e, t2tq)
    return out
